# baseline (speedup 1.0000x reference)
import functools

import jax
import jax.numpy as jnp
from jax.experimental import pallas as pl
from jax.experimental.pallas import tpu as pltpu

K = 16
NEG_INF = float("-inf")


def _graph_block_kernel(rows_ref, all_ref, out_ref, *, blk_r, n):
    h_rows = rows_ref[0]
    h_all = all_ref[0]

    def _norm(x):
        nrm = jnp.sqrt(jnp.sum(x * x, axis=-1, keepdims=True))
        return x / jnp.maximum(nrm, 1e-12)

    hn_rows = _norm(h_rows)
    hn_all = _norm(h_all)

    sim = jax.lax.dot_general(
        hn_rows, hn_all, (((1,), (1,)), ((), ())),
        preferred_element_type=jnp.float32)

    colid = jax.lax.broadcasted_iota(
        jnp.int32, (blk_r, n), 1).astype(jnp.float32)
    r0 = (pl.program_id(1) * blk_r).astype(jnp.float32)
    rowid = r0 + jax.lax.broadcasted_iota(
        jnp.int32, (blk_r, n), 0).astype(jnp.float32)
    deg = float(K + 1) + 1e-8
    dis = 1.0 / (deg ** 0.5)
    scale = dis * dis

    s = sim
    for _ in range(K):
        m = jnp.max(s, axis=-1, keepdims=True)
        s = jnp.where(s == m, NEG_INF, s)

    out0 = jnp.where(s == NEG_INF, jnp.float32(scale), jnp.float32(0.0))
    out_ref[0] = jnp.where(colid == rowid, out0 + scale, out0)

    cnt = jnp.sum(out0, axis=-1, keepdims=True)
    ties = jnp.max(cnt) > (float(K) + 0.5) * scale

    @pl.when(ties)
    def _exact_fixup():
        s2 = sim
        for _ in range(K):
            m2 = jnp.max(s2, axis=-1, keepdims=True)
            cand = jnp.where(s2 == m2, colid, float(n))
            idx = jnp.min(cand, axis=-1, keepdims=True)
            s2 = jnp.where(colid == idx, NEG_INF, s2)
        o2 = jnp.where(s2 == NEG_INF, jnp.float32(scale), jnp.float32(0.0))
        out_ref[0] = jnp.where(colid == rowid, o2 + scale, o2)


@jax.jit
def kernel(H):
    B, N, F = H.shape
    blk_r = 1024
    grid = (B, N // blk_r)
    fn = functools.partial(_graph_block_kernel, blk_r=blk_r, n=N)
    return pl.pallas_call(
        fn,
        grid=grid,
        in_specs=[
            pl.BlockSpec((1, blk_r, F), lambda b, r: (b, r, 0)),
            pl.BlockSpec((1, N, F), lambda b, r: (b, 0, 0)),
        ],
        out_specs=pl.BlockSpec((1, blk_r, N), lambda b, r: (b, r, 0)),
        out_shape=jax.ShapeDtypeStruct((B, N, N), jnp.float32),
        compiler_params=pltpu.CompilerParams(
            dimension_semantics=("parallel", "parallel")),
    )(H, H)


# speedup vs baseline: 1.2866x; 1.2866x over previous
import functools

import jax
import jax.numpy as jnp
from jax.experimental import pallas as pl
from jax.experimental.pallas import tpu as pltpu

K = 16
NEG_INF = float("-inf")


def _graph_block_kernel(rows_ref, all_ref, out_ref, *, blk_r, n):
    h_rows = rows_ref[0]
    h_all = all_ref[0]

    def _norm(x):
        nrm = jnp.sqrt(jnp.sum(x * x, axis=-1, keepdims=True))
        return x / jnp.maximum(nrm, 1e-12)

    hn_rows = _norm(h_rows)
    hn_all = _norm(h_all)

    sim = jax.lax.dot_general(
        hn_rows, hn_all, (((1,), (1,)), ((), ())),
        preferred_element_type=jnp.float32)

    colid = jax.lax.broadcasted_iota(
        jnp.int32, (blk_r, n), 1).astype(jnp.float32)
    r0 = (pl.program_id(1) * blk_r).astype(jnp.float32)
    rowid = r0 + jax.lax.broadcasted_iota(
        jnp.int32, (blk_r, n), 0).astype(jnp.float32)
    deg = float(K + 1) + 1e-8
    dis = 1.0 / (deg ** 0.5)
    scale = dis * dis

    s = sim
    for _ in range(K - 1):
        m = jnp.max(s, axis=-1, keepdims=True)
        s = jnp.where(s == m, NEG_INF, s)
    mk = jnp.max(s, axis=-1, keepdims=True)

    out0 = jnp.where(sim >= mk, jnp.float32(scale), jnp.float32(0.0))
    out_ref[0] = jnp.where(colid == rowid, out0 + scale, out0)

    cnt = jnp.sum(out0, axis=-1, keepdims=True)
    ties = jnp.max(cnt) > (float(K) + 0.5) * scale

    @pl.when(ties)
    def _exact_fixup():
        s2 = sim
        for _ in range(K):
            m2 = jnp.max(s2, axis=-1, keepdims=True)
            cand = jnp.where(s2 == m2, colid, float(n))
            idx = jnp.min(cand, axis=-1, keepdims=True)
            s2 = jnp.where(colid == idx, NEG_INF, s2)
        o2 = jnp.where(s2 == NEG_INF, jnp.float32(scale), jnp.float32(0.0))
        out_ref[0] = jnp.where(colid == rowid, o2 + scale, o2)


@jax.jit
def kernel(H):
    B, N, F = H.shape
    blk_r = 512
    grid = (B, N // blk_r)
    fn = functools.partial(_graph_block_kernel, blk_r=blk_r, n=N)
    return pl.pallas_call(
        fn,
        grid=grid,
        in_specs=[
            pl.BlockSpec((1, blk_r, F), lambda b, r: (b, r, 0)),
            pl.BlockSpec((1, N, F), lambda b, r: (b, 0, 0)),
        ],
        out_specs=pl.BlockSpec((1, blk_r, N), lambda b, r: (b, r, 0)),
        out_shape=jax.ShapeDtypeStruct((B, N, N), jnp.float32),
        compiler_params=pltpu.CompilerParams(
            dimension_semantics=("parallel", "parallel")),
    )(H, H)
